# baseline (speedup 1.0000x reference)
import functools

import jax
import jax.numpy as jnp
from jax import lax
from jax.experimental import pallas as pl
from jax.experimental.pallas import tpu as pltpu
from jax.experimental.pallas import tpu_sc as plsc

N_NODES = 100000
D_FEAT = 128
NUM_TYPES = 64

NC = 2
NS = 16
NW = NC * NS

C = 400
NCHUNKS = N_NODES // C
IDXW = 100
NGATHER = C // IDXW

_mesh = plsc.VectorSubcoreMesh(core_axis_name="c", subcore_axis_name="s")


@functools.partial(
    pl.kernel,
    out_type=jax.ShapeDtypeStruct((N_NODES, D_FEAT), jnp.float32),
    mesh=_mesh,
    compiler_params=pltpu.CompilerParams(needs_layout_passes=False),
    scratch_types=[
        pltpu.VMEM_SHARED((NUM_TYPES, D_FEAT), jnp.float32),
        pltpu.VMEM((C, D_FEAT), jnp.float32),
        pltpu.VMEM((C, D_FEAT), jnp.float32),
        pltpu.VMEM((NGATHER, IDXW), jnp.int32),
    ],
)
def _sc_embed_add(x_hbm, types_hbm, table_hbm, out_hbm,
                  table_v, xbuf, rowbuf, tbuf):
    wid = lax.axis_index("s") * NC + lax.axis_index("c")

    @pl.when(lax.axis_index("s") == 0)
    def _stage_table():
        pltpu.sync_copy(table_hbm, table_v)

    plsc.subcore_barrier()

    n_my = (NCHUNKS - wid + NW - 1) // NW

    def chunk_body(i, carry):
        c = wid + i * NW
        base = c * C
        pltpu.sync_copy(x_hbm.at[pl.ds(base, C), :], xbuf)
        pltpu.sync_copy(types_hbm.at[c], tbuf)
        for j in range(NGATHER):
            pltpu.sync_copy(
                table_v.at[tbuf.at[j]],
                rowbuf.at[pl.ds(j * IDXW, IDXW), :],
            )

        def add_body(r, carry2):
            for j in range(0, D_FEAT, 16):
                vals = rowbuf[r, pl.ds(j, 16)]
                plsc.addupdate(xbuf.at[r, pl.ds(j, 16)], vals)
            return carry2

        lax.fori_loop(0, C, add_body, 0)
        pltpu.sync_copy(xbuf, out_hbm.at[pl.ds(base, C), :])
        return carry

    lax.fori_loop(0, n_my, chunk_body, 0)


def kernel(x, node_types, type_table):
    types_r = node_types.astype(jnp.int32).reshape(NCHUNKS, NGATHER, IDXW)
    return _sc_embed_add(x, types_r, type_table)
